# baseline (speedup 1.0000x reference)
import functools

import jax
import jax.numpy as jnp
from jax import lax
from jax.experimental import pallas as pl
from jax.experimental.pallas import tpu as pltpu
from jax.experimental.pallas import tpu_sc as plsc

_START_WARM = 20000
_TOP_P = 0.15

_NW = 32
_NBINS = 65536


def _ce_body(nc, x_ref, t_ref, o_ref, p_ref):
    t = t_ref[0]
    x0 = x_ref[0, 0]
    m = x0
    picked = jnp.where(t == 0, x0, 0.0)
    for c in range(1, nc):
        xc = x_ref[0, c]
        m = jnp.maximum(m, xc)
        picked += jnp.where(t == c, xc, 0.0)
    s = jnp.zeros_like(m)
    for c in range(nc):
        s += jnp.exp(x_ref[0, c] - m)
    ce = (m + jnp.log(s)) - picked
    o_ref[0] = ce
    p_ref[...] = jnp.sum(ce).reshape(1, 1, 1, 1)


def _per_pixel_ce(inp, target, b0, nb):
    B, C, H, W = inp.shape
    R = 192
    return pl.pallas_call(
        functools.partial(_ce_body, C),
        grid=(nb, H // R),
        compiler_params=pltpu.CompilerParams(
            dimension_semantics=("parallel", "parallel")),
        in_specs=[
            pl.BlockSpec((1, C, R, W), lambda b, r: (b + b0, 0, r, 0)),
            pl.BlockSpec((1, R, W), lambda b, r: (b + b0, r, 0)),
        ],
        out_specs=[
            pl.BlockSpec((1, R, W), lambda b, r: (b, r, 0)),
            pl.BlockSpec((1, 1, 1, 1), lambda b, r: (b, r, 0, 0)),
        ],
        out_shape=[
            jax.ShapeDtypeStruct((nb, H, W), jnp.float32),
            jax.ShapeDtypeStruct((nb, H // R, 1, 1), jnp.float32),
        ],
    )(inp, target)


def _sc_hist1(n, imgs):
    chunk = n // _NW
    spi = _NW // imgs
    mesh = plsc.VectorSubcoreMesh(core_axis_name="c", subcore_axis_name="s")

    @functools.partial(
        pl.kernel,
        out_type=jax.ShapeDtypeStruct((_NW, 256, 128), jnp.int32),
        mesh=mesh,
        compiler_params=pltpu.CompilerParams(needs_layout_passes=False),
        scratch_types=[
            pltpu.VMEM((chunk // 384, 384), jnp.float32),
            pltpu.VMEM((256, 128), jnp.int32),
            pltpu.SemaphoreType.DMA,
        ],
    )
    def hist1(ce_hbm, hist_hbm, buf, hist, dsem):
        wid = lax.axis_index("s") * 2 + lax.axis_index("c")
        rows = chunk // 384
        img = wid // spi
        r0 = (wid % spi) * rows
        cp = pltpu.make_async_copy(ce_hbm.at[img, pl.ds(r0, rows)], buf, dsem)
        cp.start()
        zi = jnp.zeros((16,), jnp.int32)

        @plsc.parallel_loop(0, 256 * 128 // 16, unroll=8)
        def _zero(i):
            hist[i >> 3, pl.ds((i & 7) * 16, 16)] = zi

        cp.wait()

        ones = jnp.ones((16,), jnp.int32)

        @plsc.parallel_loop(0, rows, unroll=2)
        def _body(rr):
            for j in range(384 // 16):
                v = jnp.maximum(buf[rr, pl.ds(j * 16, 16)], 0.0)
                bits = lax.bitcast_convert_type(v, jnp.int32)
                row = lax.shift_right_logical(bits, 23)
                col = jnp.bitwise_and(lax.shift_right_logical(bits, 16), 127)
                plsc.addupdate_scatter(hist, [row, col], ones)

        pltpu.sync_copy(hist, hist_hbm.at[wid])

    return hist1


def _sc_hist2(n):
    chunk = n // _NW
    mesh = plsc.VectorSubcoreMesh(core_axis_name="c", subcore_axis_name="s")

    @functools.partial(
        pl.kernel,
        out_type=[
            jax.ShapeDtypeStruct((_NW, 512, 128), jnp.int32),
            jax.ShapeDtypeStruct((_NW, 16), jnp.float32),
        ],
        mesh=mesh,
        compiler_params=pltpu.CompilerParams(needs_layout_passes=False),
        scratch_types=[
            pltpu.VMEM((chunk // 384, 384), jnp.float32),
            pltpu.VMEM((512, 128), jnp.int32),
            pltpu.VMEM((16,), jnp.float32),
            pltpu.VMEM((16,), jnp.int32),
            pltpu.SemaphoreType.DMA,
        ],
    )
    def hist2(cea_hbm, ceb_hbm, b_hbm, hist_hbm, s1_hbm,
              buf, hist, accv, bv, dsem):
        wid = lax.axis_index("s") * 2 + lax.axis_index("c")
        rows = chunk // 384
        half = rows // 2
        img = wid >> 3
        r0 = (wid & 7) * half
        cpa = pltpu.make_async_copy(
            cea_hbm.at[img, pl.ds(r0, half)], buf.at[pl.ds(0, half)], dsem)
        cpa.start()
        cpb = pltpu.make_async_copy(
            ceb_hbm.at[img, pl.ds(r0, half)], buf.at[pl.ds(half, half)], dsem)
        cpb.start()
        pltpu.sync_copy(b_hbm.at[pl.ds(0, 16)], bv)
        zi = jnp.zeros((16,), jnp.int32)

        @plsc.parallel_loop(0, _NBINS // 16, unroll=8)
        def _zero(i):
            hist[i >> 3, pl.ds((i & 7) * 16, 16)] = zi

        cpa.wait()
        cpb.wait()

        vb = bv[...]
        ones = jnp.ones((16,), jnp.int32)
        zf = jnp.zeros((16,), jnp.float32)

        def step(rr, j, a):
            v = jnp.maximum(buf[rr, pl.ds(j * 16, 16)], 0.0)
            bits = lax.bitcast_convert_type(v, jnp.int32)
            hi = lax.shift_right_logical(bits, 16)
            row = jnp.bitwise_and(lax.shift_right_logical(bits, 7), 511)
            col = jnp.bitwise_and(bits, 127)
            plsc.addupdate_scatter(hist, [row, col], ones, mask=hi == vb)
            return a + jnp.where(hi > vb, v, 0.0)

        @plsc.parallel_loop(0, rows, carry=(zf, zf, zf, zf))
        def accs(rr, c):
            accs4 = list(c)
            for j in range(384 // 16):
                accs4[j & 3] = step(rr, j, accs4[j & 3])
            return tuple(accs4)

        a0, a1, a2, a3 = accs
        accv[...] = (a0 + a1) + (a2 + a3)
        pltpu.sync_copy(hist, hist_hbm.at[wid])
        pltpu.sync_copy(accv, s1_hbm.at[wid])

    return hist2


def _cumsum_flat(h):
    nr = h.shape[0]
    rc = h
    for sh in (1, 2, 4, 8, 16, 32, 64):
        rc = rc + jnp.concatenate(
            [jnp.zeros((nr, sh), jnp.int32), rc[:, :-sh]], axis=1)
    rt = rc[:, 127:128]
    pr = rt
    sh = 1
    while sh < nr:
        pr = pr + jnp.concatenate(
            [jnp.zeros((sh, 1), jnp.int32), pr[:-sh]], axis=0)
        sh *= 2
    incl = rc + (pr - rt)
    ntot = jnp.max(pr)
    return incl, ntot


def _flat_idx(nr):
    return (lax.broadcasted_iota(jnp.int32, (nr, 128), 0) * 128
            + lax.broadcasted_iota(jnp.int32, (nr, 128), 1))


def _scan1_body(k, ha_ref, hb_ref, pa_ref, pb_ref, b_ref, r_ref, ts_ref):
    h = jnp.sum(ha_ref[...], axis=0) + jnp.sum(hb_ref[...], axis=0)
    incl, ntot = _cumsum_flat(h)
    suffix_ge = ntot - incl + h
    fi = _flat_idx(256)
    b = jnp.max(jnp.where(suffix_ge >= k, fi, -1))
    c1 = jnp.sum(jnp.where(fi == b, suffix_ge - h, 0))
    r = k - c1
    ts = jnp.sum(pa_ref[...]) + jnp.sum(pb_ref[...])
    b_ref[...] = jnp.full((8, 128), b, jnp.int32)
    r_ref[...] = jnp.full((8, 128), r, jnp.int32)
    ts_ref[...] = jnp.full((8, 128), ts, jnp.float32)


def _scan2_body(k, n, hist_ref, s1_ref, b_ref, r_ref, ts_ref, it_ref,
                loss_ref):
    h = jnp.sum(hist_ref[...], axis=0)
    incl, _ = _cumsum_flat(h)
    b = jnp.max(b_ref[...])
    r = jnp.max(r_ref[...])
    nb = jnp.max(incl)
    suffix_ge = nb - incl + h
    fi = _flat_idx(512)
    l = jnp.max(jnp.where(suffix_ge >= r, fi, -1))
    c2 = jnp.sum(jnp.where(fi == l, suffix_ge - h, 0))
    r2 = r - c2
    vals = lax.bitcast_convert_type(
        jnp.bitwise_or(lax.shift_left(b, 16), fi), jnp.float32)
    s2 = jnp.sum(jnp.where(fi > l, h.astype(jnp.float32) * vals, 0.0))
    t = jnp.sum(jnp.where(fi == l, vals, 0.0))
    s1 = jnp.sum(s1_ref[...])
    topk = (s1 + s2 + r2.astype(jnp.float32) * t) / k
    warm = jnp.max(ts_ref[...]) / n
    loss = jnp.where(it_ref[0] < _START_WARM, warm, topk)
    loss_ref[...] = jnp.full((8, 128), loss, jnp.float32)


def _scan1(k, ha, hb, pa, pb):
    return pl.pallas_call(
        functools.partial(_scan1_body, k),
        out_shape=[
            jax.ShapeDtypeStruct((8, 128), jnp.int32),
            jax.ShapeDtypeStruct((8, 128), jnp.int32),
            jax.ShapeDtypeStruct((8, 128), jnp.float32),
        ],
    )(ha, hb, pa, pb)


def _scan2(k, n, hist, s1, b, r, ts, it):
    return pl.pallas_call(
        functools.partial(_scan2_body, k, n),
        in_specs=[
            pl.BlockSpec(memory_space=pltpu.VMEM),
            pl.BlockSpec(memory_space=pltpu.VMEM),
            pl.BlockSpec(memory_space=pltpu.VMEM),
            pl.BlockSpec(memory_space=pltpu.VMEM),
            pl.BlockSpec(memory_space=pltpu.VMEM),
            pl.BlockSpec(memory_space=pltpu.SMEM),
        ],
        out_shape=jax.ShapeDtypeStruct((8, 128), jnp.float32),
    )(hist, s1, b, r, ts, it)


def kernel(input, target, it):
    B, C, H, W = input.shape
    n = B * H * W
    k = int(n * _TOP_P)

    ce_a, psums_a = _per_pixel_ce(input, target, 0, B // 2)
    ha = _sc_hist1(n // 2, B // 2)(ce_a)
    ce_b, psums_b = _per_pixel_ce(input, target, B // 2, B // 2)
    hb = _sc_hist1(n // 2, B // 2)(ce_b)
    b, r, ts = _scan1(k, ha, hb, psums_a, psums_b)
    hist2, s1 = _sc_hist2(n)(ce_a, ce_b, b.reshape(-1))
    itv = jnp.asarray(it, jnp.int32).reshape(1)
    loss8 = _scan2(k, n, hist2, s1, b, r, ts, itv)
    return (loss8[0, 0], _TOP_P)


# speedup vs baseline: 1.0410x; 1.0410x over previous
import functools

import jax
import jax.numpy as jnp
from jax import lax
from jax.experimental import pallas as pl
from jax.experimental.pallas import tpu as pltpu
from jax.experimental.pallas import tpu_sc as plsc

_START_WARM = 20000
_TOP_P = 0.15

_NW = 32
_NBINS = 65536


def _ce_body(nc, x_ref, t_ref, o_ref, p_ref):
    t = t_ref[0]
    x0 = x_ref[0, 0]
    m = x0
    picked = jnp.where(t == 0, x0, 0.0)
    for c in range(1, nc):
        xc = x_ref[0, c]
        m = jnp.maximum(m, xc)
        picked += jnp.where(t == c, xc, 0.0)
    s = jnp.zeros_like(m)
    for c in range(nc):
        s += jnp.exp(x_ref[0, c] - m)
    ce = (m + jnp.log(s)) - picked
    o_ref[0] = ce
    p_ref[...] = jnp.sum(ce).reshape(1, 1, 1, 1)


def _per_pixel_ce(inp, target):
    B, C, H, W = inp.shape
    R = 192
    return pl.pallas_call(
        functools.partial(_ce_body, C),
        grid=(B, H // R),
        compiler_params=pltpu.CompilerParams(
            dimension_semantics=("parallel", "parallel")),
        in_specs=[
            pl.BlockSpec((1, C, R, W), lambda b, r: (b, 0, r, 0)),
            pl.BlockSpec((1, R, W), lambda b, r: (b, r, 0)),
        ],
        out_specs=[
            pl.BlockSpec((1, R, W), lambda b, r: (b, r, 0)),
            pl.BlockSpec((1, 1, 1, 1), lambda b, r: (b, r, 0, 0)),
        ],
        out_shape=[
            jax.ShapeDtypeStruct((B, H, W), jnp.float32),
            jax.ShapeDtypeStruct((B, H // R, 1, 1), jnp.float32),
        ],
    )(inp, target)


def _sc_hist1(n):
    chunk = n // _NW
    mesh = plsc.VectorSubcoreMesh(core_axis_name="c", subcore_axis_name="s")

    @functools.partial(
        pl.kernel,
        out_type=jax.ShapeDtypeStruct((_NW, 256, 128), jnp.int32),
        mesh=mesh,
        compiler_params=pltpu.CompilerParams(needs_layout_passes=False),
        scratch_types=[
            pltpu.VMEM((chunk // 384, 384), jnp.float32),
            pltpu.VMEM((256, 128), jnp.int32),
            pltpu.SemaphoreType.DMA,
        ],
    )
    def hist1(ce_hbm, hist_hbm, buf, hist, dsem):
        wid = lax.axis_index("s") * 2 + lax.axis_index("c")
        rows = chunk // 384
        img = wid >> 2
        r0 = (wid & 3) * rows
        cp = pltpu.make_async_copy(ce_hbm.at[img, pl.ds(r0, rows)], buf, dsem)
        cp.start()
        zi = jnp.zeros((16,), jnp.int32)

        @plsc.parallel_loop(0, 256 * 128 // 16, unroll=8)
        def _zero(i):
            hist[i >> 3, pl.ds((i & 7) * 16, 16)] = zi

        cp.wait()

        ones = jnp.ones((16,), jnp.int32)

        @plsc.parallel_loop(0, rows, unroll=2)
        def _body(rr):
            for j in range(384 // 16):
                v = jnp.maximum(buf[rr, pl.ds(j * 16, 16)], 0.0)
                bits = lax.bitcast_convert_type(v, jnp.int32)
                row = lax.shift_right_logical(bits, 23)
                col = jnp.bitwise_and(lax.shift_right_logical(bits, 16), 127)
                plsc.addupdate_scatter(hist, [row, col], ones)

        pltpu.sync_copy(hist, hist_hbm.at[wid])

    return hist1


def _sc_hist2(n):
    chunk = n // _NW
    mesh = plsc.VectorSubcoreMesh(core_axis_name="c", subcore_axis_name="s")

    @functools.partial(
        pl.kernel,
        out_type=[
            jax.ShapeDtypeStruct((_NW, 512, 128), jnp.int32),
            jax.ShapeDtypeStruct((_NW, 16), jnp.float32),
        ],
        mesh=mesh,
        compiler_params=pltpu.CompilerParams(needs_layout_passes=False),
        scratch_types=[
            pltpu.VMEM((chunk // 384, 384), jnp.float32),
            pltpu.VMEM((512, 128), jnp.int32),
            pltpu.VMEM((16,), jnp.float32),
            pltpu.VMEM((16,), jnp.int32),
            pltpu.SemaphoreType.DMA,
        ],
    )
    def hist2(ce_hbm, b_hbm, hist_hbm, s1_hbm, buf, hist, accv, bv, dsem):
        wid = lax.axis_index("s") * 2 + lax.axis_index("c")
        rows = chunk // 384
        img = wid >> 2
        r0 = (wid & 3) * rows
        cp = pltpu.make_async_copy(ce_hbm.at[img, pl.ds(r0, rows)], buf, dsem)
        cp.start()
        pltpu.sync_copy(b_hbm.at[pl.ds(0, 16)], bv)
        zi = jnp.zeros((16,), jnp.int32)

        @plsc.parallel_loop(0, _NBINS // 16, unroll=8)
        def _zero(i):
            hist[i >> 3, pl.ds((i & 7) * 16, 16)] = zi

        cp.wait()

        vb = bv[...]
        ones = jnp.ones((16,), jnp.int32)
        zf = jnp.zeros((16,), jnp.float32)

        def step(rr, j, a):
            v = jnp.maximum(buf[rr, pl.ds(j * 16, 16)], 0.0)
            bits = lax.bitcast_convert_type(v, jnp.int32)
            hi = lax.shift_right_logical(bits, 16)
            row = jnp.bitwise_and(lax.shift_right_logical(bits, 7), 511)
            col = jnp.bitwise_and(bits, 127)
            plsc.addupdate_scatter(hist, [row, col], ones, mask=hi == vb)
            return a + jnp.where(hi > vb, v, 0.0)

        @plsc.parallel_loop(0, rows, carry=(zf, zf, zf, zf))
        def accs(rr, c):
            accs4 = list(c)
            for j in range(384 // 16):
                accs4[j & 3] = step(rr, j, accs4[j & 3])
            return tuple(accs4)

        a0, a1, a2, a3 = accs
        accv[...] = (a0 + a1) + (a2 + a3)
        pltpu.sync_copy(hist, hist_hbm.at[wid])
        pltpu.sync_copy(accv, s1_hbm.at[wid])

    return hist2


def _cumsum_flat(h):
    nr = h.shape[0]
    rc = h
    for sh in (1, 2, 4, 8, 16, 32, 64):
        rc = rc + jnp.concatenate(
            [jnp.zeros((nr, sh), jnp.int32), rc[:, :-sh]], axis=1)
    rt = rc[:, 127:128]
    pr = rt
    sh = 1
    while sh < nr:
        pr = pr + jnp.concatenate(
            [jnp.zeros((sh, 1), jnp.int32), pr[:-sh]], axis=0)
        sh *= 2
    incl = rc + (pr - rt)
    ntot = jnp.max(pr)
    return incl, ntot


def _flat_idx(nr):
    return (lax.broadcasted_iota(jnp.int32, (nr, 128), 0) * 128
            + lax.broadcasted_iota(jnp.int32, (nr, 128), 1))


def _scan1_body(k, hist_ref, psum_ref, b_ref, r_ref, ts_ref):
    h = jnp.sum(hist_ref[...], axis=0)
    incl, ntot = _cumsum_flat(h)
    suffix_ge = ntot - incl + h
    fi = _flat_idx(256)
    b = jnp.max(jnp.where(suffix_ge >= k, fi, -1))
    c1 = jnp.sum(jnp.where(fi == b, suffix_ge - h, 0))
    r = k - c1
    ts = jnp.sum(psum_ref[...])
    b_ref[...] = jnp.full((8, 128), b, jnp.int32)
    r_ref[...] = jnp.full((8, 128), r, jnp.int32)
    ts_ref[...] = jnp.full((8, 128), ts, jnp.float32)


def _scan2_body(k, n, hist_ref, s1_ref, b_ref, r_ref, ts_ref, it_ref,
                loss_ref):
    h = jnp.sum(hist_ref[...], axis=0)
    incl, _ = _cumsum_flat(h)
    b = jnp.max(b_ref[...])
    r = jnp.max(r_ref[...])
    nb = jnp.max(incl)
    suffix_ge = nb - incl + h
    fi = _flat_idx(512)
    l = jnp.max(jnp.where(suffix_ge >= r, fi, -1))
    c2 = jnp.sum(jnp.where(fi == l, suffix_ge - h, 0))
    r2 = r - c2
    vals = lax.bitcast_convert_type(
        jnp.bitwise_or(lax.shift_left(b, 16), fi), jnp.float32)
    s2 = jnp.sum(jnp.where(fi > l, h.astype(jnp.float32) * vals, 0.0))
    t = jnp.sum(jnp.where(fi == l, vals, 0.0))
    s1 = jnp.sum(s1_ref[...])
    topk = (s1 + s2 + r2.astype(jnp.float32) * t) / k
    warm = jnp.max(ts_ref[...]) / n
    loss = jnp.where(it_ref[0] < _START_WARM, warm, topk)
    loss_ref[...] = jnp.full((8, 128), loss, jnp.float32)


def _scan1(k, hist, psums):
    return pl.pallas_call(
        functools.partial(_scan1_body, k),
        out_shape=[
            jax.ShapeDtypeStruct((8, 128), jnp.int32),
            jax.ShapeDtypeStruct((8, 128), jnp.int32),
            jax.ShapeDtypeStruct((8, 128), jnp.float32),
        ],
    )(hist, psums)


def _scan2(k, n, hist, s1, b, r, ts, it):
    return pl.pallas_call(
        functools.partial(_scan2_body, k, n),
        in_specs=[
            pl.BlockSpec(memory_space=pltpu.VMEM),
            pl.BlockSpec(memory_space=pltpu.VMEM),
            pl.BlockSpec(memory_space=pltpu.VMEM),
            pl.BlockSpec(memory_space=pltpu.VMEM),
            pl.BlockSpec(memory_space=pltpu.VMEM),
            pl.BlockSpec(memory_space=pltpu.SMEM),
        ],
        out_shape=jax.ShapeDtypeStruct((8, 128), jnp.float32),
    )(hist, s1, b, r, ts, it)


def kernel(input, target, it):
    B, C, H, W = input.shape
    n = B * H * W
    k = int(n * _TOP_P)

    ce, psums = _per_pixel_ce(input, target)
    hist1 = _sc_hist1(n)(ce)
    b, r, ts = _scan1(k, hist1, psums)
    hist2, s1 = _sc_hist2(n)(ce, b.reshape(-1))
    itv = jnp.asarray(it, jnp.int32).reshape(1)
    loss8 = _scan2(k, n, hist2, s1, b, r, ts, itv)
    return (loss8[0, 0], _TOP_P)
